# Initial kernel scaffold; baseline (speedup 1.0000x reference)
import jax
import jax.numpy as jnp
from jax.experimental import pallas as pl


def kernel(x, edge_index, edge_attr, W1, b1, g1, beta1, W2, b2, g2, beta2, W3, b3):
    raise NotImplementedError("write your pallas kernel here")


# speedup vs baseline: 1.5988x; 1.5988x over previous
import functools

import jax
import jax.numpy as jnp
from jax import lax
from jax.experimental import pallas as pl
from jax.experimental.pallas import tpu as pltpu
from jax.experimental.pallas import tpu_sc as plsc

_N = 100000
_E = 1600000
_BN_EPS = 1e-5


def _matmul_body(x_ref, w_ref, o_ref):
    o_ref[...] = jnp.dot(x_ref[...], w_ref[...],
                         preferred_element_type=jnp.float32)


def _dense(x, W, bn):
    n, k = x.shape
    m = W.shape[1]
    return pl.pallas_call(
        _matmul_body,
        grid=(n // bn,),
        in_specs=[pl.BlockSpec((bn, k), lambda i: (i, 0)),
                  pl.BlockSpec((k, m), lambda i: (0, 0))],
        out_specs=pl.BlockSpec((bn, m), lambda i: (i, 0)),
        out_shape=jax.ShapeDtypeStruct((n, m), jnp.float32),
    )(x, W)


def _affine_relu_matmul_body(a_ref, s_ref, c_ref, w_ref, o_ref):
    h = jnp.maximum(a_ref[...] * s_ref[...] + c_ref[...], 0.0)
    o_ref[...] = jnp.dot(h, w_ref[...], preferred_element_type=jnp.float32)


def _affine_relu_matmul(a, s, c, W, bn):
    n, k = a.shape
    m = W.shape[1]
    return pl.pallas_call(
        _affine_relu_matmul_body,
        grid=(n // bn,),
        in_specs=[pl.BlockSpec((bn, k), lambda i: (i, 0)),
                  pl.BlockSpec((1, k), lambda i: (0, 0)),
                  pl.BlockSpec((1, k), lambda i: (0, 0)),
                  pl.BlockSpec((k, m), lambda i: (0, 0))],
        out_specs=pl.BlockSpec((bn, m), lambda i: (i, 0)),
        out_shape=jax.ShapeDtypeStruct((n, m), jnp.float32),
    )(a, s, c, W)


def _scale_body(g_ref, ea_ref, o_ref):
    w = jnp.mean(ea_ref[...], axis=1, keepdims=True)
    o_ref[...] = g_ref[..., : o_ref.shape[-1]] * w


def _scale_by_edge_mean(g, edge_attr, dout, be):
    e, d = g.shape
    de = edge_attr.shape[1]
    return pl.pallas_call(
        _scale_body,
        grid=(e // be,),
        in_specs=[pl.BlockSpec((be, d), lambda i: (i, 0)),
                  pl.BlockSpec((be, de), lambda i: (i, 0))],
        out_specs=pl.BlockSpec((be, dout), lambda i: (i, 0)),
        out_shape=jax.ShapeDtypeStruct((e, dout), jnp.float32),
    )(g, edge_attr)


def _bias_body(a_ref, b_ref, o_ref):
    o_ref[...] = a_ref[...] + b_ref[...]


def _bias_add(a, b, bn):
    n, m = a.shape
    return pl.pallas_call(
        _bias_body,
        grid=(n // bn,),
        in_specs=[pl.BlockSpec((bn, m), lambda i: (i, 0)),
                  pl.BlockSpec((1, m), lambda i: (0, 0))],
        out_specs=pl.BlockSpec((bn, m), lambda i: (i, 0)),
        out_shape=jax.ShapeDtypeStruct((n, m), jnp.float32),
    )(a, b)


def _sc_gather(table, idx):
    v, d = table.shape
    b = idx.shape[0]
    info = plsc.get_sparse_core_info()
    nc, ns = info.num_cores, info.num_subcores
    nw = nc * ns
    b_per_w = b // nw
    ch = 400
    assert b % nw == 0 and b_per_w % ch == 0 and ch % 8 == 0
    mesh = plsc.VectorSubcoreMesh(core_axis_name="c", subcore_axis_name="s")

    @functools.partial(
        pl.kernel, mesh=mesh,
        out_type=jax.ShapeDtypeStruct((b, d), jnp.float32),
        scratch_types=[
            pltpu.VMEM((ch,), jnp.int32),
            pltpu.VMEM((ch, d), jnp.float32),
            pltpu.SemaphoreType.DMA,
        ],
    )
    def k(table_hbm, idx_hbm, out_hbm, idx_v, rows_v, sem):
        wid = lax.axis_index("s") * nc + lax.axis_index("c")

        def body(i, carry):
            base = wid * b_per_w + i * ch
            pltpu.sync_copy(idx_hbm.at[pl.ds(base, ch)], idx_v)
            pltpu.async_copy(table_hbm.at[idx_v], rows_v, sem).wait()
            pltpu.sync_copy(rows_v, out_hbm.at[pl.ds(base, ch)])
            return carry

        lax.fori_loop(0, b_per_w // ch, body, 0)

    return k(table, idx)


def kernel(x, edge_index, edge_attr, W1, b1, g1, beta1, W2, b2, g2, beta2,
           W3, b3):
    x = x.astype(jnp.float32)
    src = edge_index[0]
    dst = edge_index[1]

    inv = 1.0 / jnp.sqrt(1.0 + _BN_EPS)
    s1 = (g1 * inv).reshape(1, -1)
    c1 = (b1 * g1 * inv + beta1).reshape(1, -1)
    s2 = (g2 * inv).reshape(1, -1)
    c2 = (b2 * g2 * inv + beta2).reshape(1, -1)

    W1p = jnp.pad(W1, ((0, 0), (0, 128 - W1.shape[1])))
    W2p = jnp.pad(W2, ((0, 0), (0, 128 - W2.shape[1])))

    h1 = _dense(x, W1p, 1000)
    m1 = _scale_by_edge_mean(_sc_gather(h1, src), edge_attr, 64, 2000)
    a1 = jax.ops.segment_sum(m1, dst, num_segments=_N)

    h2 = _affine_relu_matmul(a1, s1, c1, W2p, 1000)
    m2 = _scale_by_edge_mean(_sc_gather(h2, src), edge_attr, 64, 2000)
    a2 = jax.ops.segment_sum(m2, dst, num_segments=_N)

    h3 = _affine_relu_matmul(a2, s2, c2, W3, 1000)
    m3 = _sc_gather(h3, src)
    a3 = jax.ops.segment_sum(m3, dst, num_segments=_N)

    return _bias_add(a3, b3.reshape(1, -1), 1000)
